# baseline (speedup 1.0000x reference)
import functools

import jax
import jax.numpy as jnp
from jax import lax
from jax.experimental import pallas as pl
from jax.experimental.pallas import tpu as pltpu
from jax.experimental.pallas import tpu_sc as plsc

_DEC = 4
_K = 16
_NB = 8


def _knn_kernel(kbs_ref, kbn_ref, y_ref, xt_ref, by_ref, bx_ref, nbr_ref, *,
                qb, kb):
    i = pl.program_id(0)
    kb0 = kbs_ref[i]
    nkb = kbn_ref[i]
    y = y_ref[...]
    y0 = y[:, 0:1]
    y1 = y[:, 1:2]
    y2 = y[:, 2:3]
    byv = by_ref[...]

    topv0 = jnp.full((qb, _K), jnp.inf, jnp.float32)
    topi0 = jnp.zeros((qb, _K), jnp.int32)

    lanes16 = lax.broadcasted_iota(jnp.int32, (qb, _K), 1)

    def body(j, carry):
        topv, topi = carry
        base = pl.multiple_of(kb0 + j * kb, 128)
        xb = xt_ref[:, pl.ds(base, kb)]
        bxb = bx_ref[:, pl.ds(base, kb)]
        d = ((y0 - xb[0:1, :]) ** 2
             + (y1 - xb[1:2, :]) ** 2
             + (y2 - xb[2:3, :]) ** 2)
        d = jnp.where(byv == bxb, d, jnp.inf)

        def ext(t, c):
            d, bv, bi = c
            m = jnp.min(d, axis=1, keepdims=True)
            eq = d <= m
            lanes = lax.broadcasted_iota(jnp.int32, (qb, kb), 1)
            am = jnp.min(jnp.where(eq, lanes, 2147483647),
                         axis=1, keepdims=True)
            d = jnp.where(eq, jnp.inf, d)
            bv = jnp.where(lanes16 == t, m, bv)
            bi = jnp.where(lanes16 == t, am, bi)
            return d, bv, bi

        bv0 = jnp.full((qb, _K), jnp.inf, jnp.float32)
        bi0 = jnp.zeros((qb, _K), jnp.int32)
        _, bv, bi = lax.fori_loop(0, _K, ext, (d, bv0, bi0))
        bi = bi + base

        cv = jnp.concatenate([topv, bv], axis=1)
        ci = jnp.concatenate([topi, bi], axis=1)

        def ext2(t, c):
            cv, ci, nv, ni = c
            m = jnp.min(cv, axis=1, keepdims=True)
            eq = cv <= m
            am = jnp.min(jnp.where(eq, ci, 2147483647), axis=1, keepdims=True)
            cv = jnp.where(eq, jnp.inf, cv)
            nv = jnp.where(lanes16 == t, m, nv)
            ni = jnp.where(lanes16 == t, am, ni)
            return cv, ci, nv, ni

        _, _, nv, ni = lax.fori_loop(0, _K, ext2, (cv, ci, topv, topi))
        return nv, ni

    topv, topi = lax.fori_loop(0, nkb, body, (topv0, topi0))
    nbr_ref[...] = topi


def _knn(pos_y, batch_y, pos_x, batch_x, kb0s, nkbs, qb, kb):
    m = pos_y.shape[0]
    n = pos_x.shape[0]
    n_pad = n + kb
    xt = jnp.pad(pos_x.T, ((0, 0), (0, kb)))
    bx = jnp.pad(batch_x[None, :], ((0, 0), (0, kb)), constant_values=-1)
    kern = functools.partial(_knn_kernel, qb=qb, kb=kb)
    return pl.pallas_call(
        kern,
        grid_spec=pltpu.PrefetchScalarGridSpec(
            num_scalar_prefetch=2,
            grid=(m // qb,),
            in_specs=[
                pl.BlockSpec((qb, 3), lambda i, *_: (i, 0)),
                pl.BlockSpec((3, n_pad), lambda i, *_: (0, 0)),
                pl.BlockSpec((qb, 1), lambda i, *_: (i, 0)),
                pl.BlockSpec((1, n_pad), lambda i, *_: (0, 0)),
            ],
            out_specs=pl.BlockSpec((qb, _K), lambda i, *_: (i, 0)),
        ),
        out_shape=jax.ShapeDtypeStruct((m, _K), jnp.int32),
    )(kb0s, nkbs, pos_y, xt, batch_y[:, None], bx)


def _key_ranges(batch_y, batch_x, qb, kb):
    b_lo = batch_y[::qb]
    b_hi = batch_y[qb - 1::qb]
    seg_start = jnp.searchsorted(batch_x, b_lo, side="left")
    seg_end = jnp.searchsorted(batch_x, b_hi, side="right")
    kb0 = ((seg_start // 128) * 128).astype(jnp.int32)
    nkb = jnp.maximum((seg_end - kb0 + kb - 1) // kb, 1).astype(jnp.int32)
    return kb0, nkb


def _sc_maxgather(table, nbr, d_red):
    m, k = nbr.shape
    d = 128
    nw = 32
    mq = m // nw
    chunk_q = 128 // k
    chunk = chunk_q * k
    nchunks = mq // chunk_q
    cpr = min(4, nchunks)
    rounds = nchunks // cpr
    rq = cpr * chunk_q
    idx3 = nbr.reshape(nw, nchunks, chunk)
    mesh = plsc.VectorSubcoreMesh(core_axis_name="c", subcore_axis_name="s")

    @functools.partial(
        pl.kernel, mesh=mesh,
        out_type=jax.ShapeDtypeStruct((m, d), jnp.float32),
        scratch_types=[
            pltpu.VMEM((nchunks, chunk), jnp.int32),
            pltpu.VMEM((cpr * chunk, d), jnp.float32),
            pltpu.VMEM((mq, d), jnp.float32),
            pltpu.SemaphoreType.DMA,
        ],
    )
    def gather_kernel(table_hbm, idx_hbm, out_hbm, idx_v, rows_v, out_v, sem):
        wid = lax.axis_index("s") * 2 + lax.axis_index("c")
        pltpu.sync_copy(idx_hbm.at[wid], idx_v)
        for r in range(rounds):
            copies = [pltpu.async_copy(
                table_hbm.at[idx_v.at[r * cpr + g]],
                rows_v.at[pl.ds(g * chunk, chunk)],
                sem) for g in range(cpr)]
            for c in copies:
                c.wait()

            def qbody(q, carry, r=r):
                for c0 in range(d_red // 16):
                    acc = rows_v[q * k, pl.ds(c0 * 16, 16)]
                    for j in range(1, k):
                        acc = jnp.maximum(acc,
                                          rows_v[q * k + j, pl.ds(c0 * 16, 16)])
                    out_v[r * rq + q, pl.ds(c0 * 16, 16)] = acc
                return carry

            lax.fori_loop(0, rq, qbody, 0)
        pltpu.sync_copy(out_v, out_hbm.at[pl.ds(wid * mq, mq)])

    return gather_kernel(table, idx3)


def _mm3(p, w):
    return p[:, 0:1] * w[0:1, :] + p[:, 1:2] * w[1:2, :] + p[:, 2:3] * w[2:3, :]


def _prep1_kernel(pos_ref, w1_ref, p_ref):
    p_ref[...] = _mm3(pos_ref[...], w1_ref[...])


def _prep2_kernel(g1_ref, pos1_ref, pos2_ref, w1_ref, b1_ref, w2a_ref,
                  w2b_ref, b2_ref, q_ref, c2_ref):
    pos1 = pos1_ref[...]
    x1 = g1_ref[...] + (b1_ref[...] - _mm3(pos1, w1_ref[...]))
    w2b = w2b_ref[...]
    r = _mm3(pos1, w2b)
    q_ref[...] = jnp.dot(x1, w2a_ref[...],
                         preferred_element_type=jnp.float32) + r
    c2_ref[...] = b2_ref[...] - _mm3(pos2_ref[...], w2b)


def _final_kernel(g2_ref, c2_ref, pos2_ref, bm_ref, w3ax_ref, w3ap_ref,
                  b3a_ref, w3b_ref, b3b_ref, w3c_ref, b3c_ref, w4a_ref,
                  b4a_ref, w4b_ref, b4b_ref, w4c_ref, b4c_ref, out_ref):
    x2 = g2_ref[...] + c2_ref[...]
    hp = _mm3(pos2_ref[...], w3ap_ref[...])
    h = jnp.maximum(jnp.dot(x2, w3ax_ref[...],
                            preferred_element_type=jnp.float32)
                    + hp + b3a_ref[...], 0.0)
    h = jnp.maximum(jnp.dot(h, w3b_ref[...],
                            preferred_element_type=jnp.float32)
                    + b3b_ref[...], 0.0)
    h = jnp.dot(h, w3c_ref[...], preferred_element_type=jnp.float32) \
        + b3c_ref[...]
    bm = bm_ref[...]
    rows = [jnp.max(jnp.where(bm == b, h, -jnp.inf), axis=0, keepdims=True)
            for b in range(_NB)]
    g = jnp.concatenate(rows, axis=0)
    o = jnp.maximum(jnp.dot(g, w4a_ref[...],
                            preferred_element_type=jnp.float32)
                    + b4a_ref[...], 0.0)
    o = jnp.maximum(jnp.dot(o, w4b_ref[...],
                            preferred_element_type=jnp.float32)
                    + b4b_ref[...], 0.0)
    o = jnp.dot(o, w4c_ref[...], preferred_element_type=jnp.float32) \
        + b4c_ref[...]
    e = o - jnp.max(o, axis=1, keepdims=True)
    out_ref[...] = e - jnp.log(jnp.sum(jnp.exp(e), axis=1, keepdims=True))


def _call(kern, out_shapes, *args):
    return pl.pallas_call(kern, out_shape=out_shapes)(*args)


def kernel(pos, batch, W1, b1, W2, b2, W3a, b3a, W3b, b3b, W3c, b3c,
           W4a, b4a, W4b, b4b, W4c, b4c):
    batch = batch.astype(jnp.int32)
    pos1 = pos[::_DEC]
    batch1 = batch[::_DEC]
    pos2 = pos1[::_DEC]
    batch2 = batch1[::_DEC]

    kb0s1, nkbs1 = _key_ranges(batch1, batch, qb=256, kb=2304)
    nbr1 = _knn(pos1, batch1, pos, batch, kb0s1, nkbs1, qb=256, kb=2304)
    w1p = jnp.pad(W1, ((0, 0), (0, 96)))
    p_feat = _call(_prep1_kernel, jax.ShapeDtypeStruct((pos.shape[0], 128),
                                                       jnp.float32), pos, w1p)
    g1 = _sc_maxgather(p_feat, nbr1, d_red=32)[:, :32]

    q_feat, c2 = _call(
        _prep2_kernel,
        (jax.ShapeDtypeStruct((pos1.shape[0], 128), jnp.float32),
         jax.ShapeDtypeStruct((pos2.shape[0], 128), jnp.float32)),
        g1, pos1, pos2, W1, b1[None, :], W2[:32], W2[32:], b2[None, :])

    kb0s2, nkbs2 = _key_ranges(batch2, batch1, qb=256, kb=768)
    nbr2 = _knn(pos2, batch2, pos1, batch1, kb0s2, nkbs2, qb=256, kb=768)
    g2 = _sc_maxgather(q_feat, nbr2, d_red=128)

    return _call(
        _final_kernel, jax.ShapeDtypeStruct((_NB, 10), jnp.float32),
        g2, c2, pos2, batch2[:, None], W3a[:128], W3a[128:], b3a[None, :],
        W3b, b3b[None, :], W3c, b3c[None, :], W4a, b4a[None, :],
        W4b, b4b[None, :], W4c, b4c[None, :])


# speedup vs baseline: 1.2045x; 1.2045x over previous
import functools

import jax
import jax.numpy as jnp
from jax import lax
from jax.experimental import pallas as pl
from jax.experimental.pallas import tpu as pltpu
from jax.experimental.pallas import tpu_sc as plsc

_DEC = 4
_K = 16
_NB = 8


def _knn_kernel(kbs_ref, kbn_ref, y_ref, xt_ref, by_ref, bx_ref, nbr_ref, *,
                qb, kb):
    i = pl.program_id(0)
    kb0 = kbs_ref[i]
    nkb = kbn_ref[i]
    y = y_ref[...]
    y0 = y[:, 0:1]
    y1 = y[:, 1:2]
    y2 = y[:, 2:3]
    byv = by_ref[...]

    topv0 = jnp.full((qb, _K), jnp.inf, jnp.float32)
    topi0 = jnp.zeros((qb, _K), jnp.int32)

    lanes16 = lax.broadcasted_iota(jnp.int32, (qb, _K), 1)

    def body(j, carry):
        topv, topi = carry
        base = pl.multiple_of(kb0 + j * kb, 128)
        xb = xt_ref[:, pl.ds(base, kb)]
        bxb = bx_ref[:, pl.ds(base, kb)]
        d = ((y0 - xb[0:1, :]) ** 2
             + (y1 - xb[1:2, :]) ** 2
             + (y2 - xb[2:3, :]) ** 2)
        d = jnp.where(byv == bxb, d, jnp.inf)
        gidx = base + lax.broadcasted_iota(jnp.int32, (qb, kb), 1)

        cv = jnp.concatenate([topv, d], axis=1)
        ci = jnp.concatenate([topi, gidx], axis=1)

        def ext(t, c):
            cv, ci, nv, ni = c
            m = jnp.min(cv, axis=1, keepdims=True)
            eq = cv <= m
            am = jnp.min(jnp.where(eq, ci, 2147483647), axis=1, keepdims=True)
            cv = jnp.where(eq, jnp.inf, cv)
            nv = jnp.where(lanes16 == t, m, nv)
            ni = jnp.where(lanes16 == t, am, ni)
            return cv, ci, nv, ni

        _, _, nv, ni = lax.fori_loop(0, _K, ext, (cv, ci, topv, topi))
        return nv, ni

    topv, topi = lax.fori_loop(0, nkb, body, (topv0, topi0))
    nbr_ref[...] = topi


def _knn(pos_y, batch_y, pos_x, batch_x, kb0s, nkbs, qb, kb):
    m = pos_y.shape[0]
    n = pos_x.shape[0]
    n_pad = n + kb
    xt = jnp.pad(pos_x.T, ((0, 0), (0, kb)))
    bx = jnp.pad(batch_x[None, :], ((0, 0), (0, kb)), constant_values=-1)
    kern = functools.partial(_knn_kernel, qb=qb, kb=kb)
    return pl.pallas_call(
        kern,
        grid_spec=pltpu.PrefetchScalarGridSpec(
            num_scalar_prefetch=2,
            grid=(m // qb,),
            in_specs=[
                pl.BlockSpec((qb, 3), lambda i, *_: (i, 0)),
                pl.BlockSpec((3, n_pad), lambda i, *_: (0, 0)),
                pl.BlockSpec((qb, 1), lambda i, *_: (i, 0)),
                pl.BlockSpec((1, n_pad), lambda i, *_: (0, 0)),
            ],
            out_specs=pl.BlockSpec((qb, _K), lambda i, *_: (i, 0)),
        ),
        out_shape=jax.ShapeDtypeStruct((m, _K), jnp.int32),
    )(kb0s, nkbs, pos_y, xt, batch_y[:, None], bx)


def _key_ranges(batch_y, batch_x, qb, kb):
    b_lo = batch_y[::qb]
    b_hi = batch_y[qb - 1::qb]
    seg_start = jnp.searchsorted(batch_x, b_lo, side="left")
    seg_end = jnp.searchsorted(batch_x, b_hi, side="right")
    kb0 = ((seg_start // 128) * 128).astype(jnp.int32)
    nkb = jnp.maximum((seg_end - kb0 + kb - 1) // kb, 1).astype(jnp.int32)
    return kb0, nkb


def _sc_maxgather(table, nbr, d_red):
    m, k = nbr.shape
    d = 128
    nw = 32
    mq = m // nw
    chunk_q = 128 // k
    chunk = chunk_q * k
    nchunks = mq // chunk_q
    cpr = min(4, nchunks)
    rounds = nchunks // cpr
    rq = cpr * chunk_q
    idx3 = nbr.reshape(nw, nchunks, chunk)
    mesh = plsc.VectorSubcoreMesh(core_axis_name="c", subcore_axis_name="s")

    @functools.partial(
        pl.kernel, mesh=mesh,
        out_type=jax.ShapeDtypeStruct((m, d), jnp.float32),
        scratch_types=[
            pltpu.VMEM((nchunks, chunk), jnp.int32),
            pltpu.VMEM((cpr * chunk, d), jnp.float32),
            pltpu.VMEM((mq, d), jnp.float32),
            pltpu.SemaphoreType.DMA,
        ],
    )
    def gather_kernel(table_hbm, idx_hbm, out_hbm, idx_v, rows_v, out_v, sem):
        wid = lax.axis_index("s") * 2 + lax.axis_index("c")
        pltpu.sync_copy(idx_hbm.at[wid], idx_v)
        for r in range(rounds):
            copies = [pltpu.async_copy(
                table_hbm.at[idx_v.at[r * cpr + g]],
                rows_v.at[pl.ds(g * chunk, chunk)],
                sem) for g in range(cpr)]
            for c in copies:
                c.wait()

            def qbody(q, carry, r=r):
                for c0 in range(d_red // 16):
                    acc = rows_v[q * k, pl.ds(c0 * 16, 16)]
                    for j in range(1, k):
                        acc = jnp.maximum(acc,
                                          rows_v[q * k + j, pl.ds(c0 * 16, 16)])
                    out_v[r * rq + q, pl.ds(c0 * 16, 16)] = acc
                return carry

            lax.fori_loop(0, rq, qbody, 0)
        pltpu.sync_copy(out_v, out_hbm.at[pl.ds(wid * mq, mq)])

    return gather_kernel(table, idx3)


def _mm3(p, w):
    return p[:, 0:1] * w[0:1, :] + p[:, 1:2] * w[1:2, :] + p[:, 2:3] * w[2:3, :]


def _prep1_kernel(pos_ref, w1_ref, p_ref):
    p_ref[...] = _mm3(pos_ref[...], w1_ref[...])


def _prep2_kernel(g1_ref, pos1_ref, pos2_ref, w1_ref, b1_ref, w2a_ref,
                  w2b_ref, b2_ref, q_ref, c2_ref):
    pos1 = pos1_ref[...]
    x1 = g1_ref[...] + (b1_ref[...] - _mm3(pos1, w1_ref[...]))
    w2b = w2b_ref[...]
    r = _mm3(pos1, w2b)
    q_ref[...] = jnp.dot(x1, w2a_ref[...],
                         preferred_element_type=jnp.float32) + r
    c2_ref[...] = b2_ref[...] - _mm3(pos2_ref[...], w2b)


def _final_kernel(g2_ref, c2_ref, pos2_ref, bm_ref, w3ax_ref, w3ap_ref,
                  b3a_ref, w3b_ref, b3b_ref, w3c_ref, b3c_ref, w4a_ref,
                  b4a_ref, w4b_ref, b4b_ref, w4c_ref, b4c_ref, out_ref):
    x2 = g2_ref[...] + c2_ref[...]
    hp = _mm3(pos2_ref[...], w3ap_ref[...])
    h = jnp.maximum(jnp.dot(x2, w3ax_ref[...],
                            preferred_element_type=jnp.float32)
                    + hp + b3a_ref[...], 0.0)
    h = jnp.maximum(jnp.dot(h, w3b_ref[...],
                            preferred_element_type=jnp.float32)
                    + b3b_ref[...], 0.0)
    h = jnp.dot(h, w3c_ref[...], preferred_element_type=jnp.float32) \
        + b3c_ref[...]
    bm = bm_ref[...]
    rows = [jnp.max(jnp.where(bm == b, h, -jnp.inf), axis=0, keepdims=True)
            for b in range(_NB)]
    g = jnp.concatenate(rows, axis=0)
    o = jnp.maximum(jnp.dot(g, w4a_ref[...],
                            preferred_element_type=jnp.float32)
                    + b4a_ref[...], 0.0)
    o = jnp.maximum(jnp.dot(o, w4b_ref[...],
                            preferred_element_type=jnp.float32)
                    + b4b_ref[...], 0.0)
    o = jnp.dot(o, w4c_ref[...], preferred_element_type=jnp.float32) \
        + b4c_ref[...]
    e = o - jnp.max(o, axis=1, keepdims=True)
    out_ref[...] = e - jnp.log(jnp.sum(jnp.exp(e), axis=1, keepdims=True))


def _call(kern, out_shapes, *args):
    return pl.pallas_call(kern, out_shape=out_shapes)(*args)


def kernel(pos, batch, W1, b1, W2, b2, W3a, b3a, W3b, b3b, W3c, b3c,
           W4a, b4a, W4b, b4b, W4c, b4c):
    batch = batch.astype(jnp.int32)
    pos1 = pos[::_DEC]
    batch1 = batch[::_DEC]
    pos2 = pos1[::_DEC]
    batch2 = batch1[::_DEC]

    kb0s1, nkbs1 = _key_ranges(batch1, batch, qb=256, kb=2304)
    nbr1 = _knn(pos1, batch1, pos, batch, kb0s1, nkbs1, qb=256, kb=2304)
    w1p = jnp.pad(W1, ((0, 0), (0, 96)))
    p_feat = _call(_prep1_kernel, jax.ShapeDtypeStruct((pos.shape[0], 128),
                                                       jnp.float32), pos, w1p)
    g1 = _sc_maxgather(p_feat, nbr1, d_red=32)[:, :32]

    q_feat, c2 = _call(
        _prep2_kernel,
        (jax.ShapeDtypeStruct((pos1.shape[0], 128), jnp.float32),
         jax.ShapeDtypeStruct((pos2.shape[0], 128), jnp.float32)),
        g1, pos1, pos2, W1, b1[None, :], W2[:32], W2[32:], b2[None, :])

    kb0s2, nkbs2 = _key_ranges(batch2, batch1, qb=256, kb=768)
    nbr2 = _knn(pos2, batch2, pos1, batch1, kb0s2, nkbs2, qb=256, kb=768)
    g2 = _sc_maxgather(q_feat, nbr2, d_red=128)

    return _call(
        _final_kernel, jax.ShapeDtypeStruct((_NB, 10), jnp.float32),
        g2, c2, pos2, batch2[:, None], W3a[:128], W3a[128:], b3a[None, :],
        W3b, b3b[None, :], W3c, b3c[None, :], W4a, b4a[None, :],
        W4b, b4b[None, :], W4c, b4c[None, :])


# speedup vs baseline: 1.4483x; 1.2024x over previous
import functools

import jax
import jax.numpy as jnp
from jax import lax
from jax.experimental import pallas as pl
from jax.experimental.pallas import tpu as pltpu
from jax.experimental.pallas import tpu_sc as plsc

_DEC = 4
_K = 16
_NB = 8


def _knn_kernel(kbs_ref, kbn_ref, y_ref, xt_ref, by_ref, bx_ref, nbr_ref, *,
                qb, kb):
    i = pl.program_id(0)
    kb0 = kbs_ref[i]
    nkb = kbn_ref[i]
    y = y_ref[...]
    y0 = y[:, 0:1]
    y1 = y[:, 1:2]
    y2 = y[:, 2:3]
    byv = by_ref[...]

    topv0 = jnp.full((qb, _K), jnp.inf, jnp.float32)
    topi0 = jnp.zeros((qb, _K), jnp.int32)

    lanes16 = lax.broadcasted_iota(jnp.int32, (qb, _K), 1)

    def body(j, carry):
        topv, topi = carry
        base = pl.multiple_of(kb0 + j * kb, 128)
        xb = xt_ref[:, pl.ds(base, kb)]
        bxb = bx_ref[:, pl.ds(base, kb)]
        d = ((y0 - xb[0:1, :]) ** 2
             + (y1 - xb[1:2, :]) ** 2
             + (y2 - xb[2:3, :]) ** 2)
        d = jnp.where(byv == bxb, d, jnp.inf)
        gidx = base + lax.broadcasted_iota(jnp.int32, (qb, kb), 1)

        cv = jnp.concatenate([topv, d], axis=1)
        ci = jnp.concatenate([topi, gidx], axis=1)

        def ext(t, c):
            m_prev, nv, ni = c
            m = jnp.min(jnp.where(cv > m_prev, cv, jnp.inf),
                        axis=1, keepdims=True)
            am = jnp.min(jnp.where(cv == m, ci, 2147483647),
                         axis=1, keepdims=True)
            nv = jnp.where(lanes16 == t, m, nv)
            ni = jnp.where(lanes16 == t, am, ni)
            return m, nv, ni

        m0 = jnp.full((qb, 1), -jnp.inf, jnp.float32)
        _, nv, ni = lax.fori_loop(0, _K, ext, (m0, topv, topi))
        return nv, ni

    topv, topi = lax.fori_loop(0, nkb, body, (topv0, topi0))
    nbr_ref[...] = topi


def _knn(pos_y, batch_y, pos_x, batch_x, kb0s, nkbs, qb, kb):
    m = pos_y.shape[0]
    n = pos_x.shape[0]
    n_pad = n + kb
    xt = jnp.pad(pos_x.T, ((0, 0), (0, kb)))
    bx = jnp.pad(batch_x[None, :], ((0, 0), (0, kb)), constant_values=-1)
    kern = functools.partial(_knn_kernel, qb=qb, kb=kb)
    return pl.pallas_call(
        kern,
        grid_spec=pltpu.PrefetchScalarGridSpec(
            num_scalar_prefetch=2,
            grid=(m // qb,),
            in_specs=[
                pl.BlockSpec((qb, 3), lambda i, *_: (i, 0)),
                pl.BlockSpec((3, n_pad), lambda i, *_: (0, 0)),
                pl.BlockSpec((qb, 1), lambda i, *_: (i, 0)),
                pl.BlockSpec((1, n_pad), lambda i, *_: (0, 0)),
            ],
            out_specs=pl.BlockSpec((qb, _K), lambda i, *_: (i, 0)),
        ),
        out_shape=jax.ShapeDtypeStruct((m, _K), jnp.int32),
    )(kb0s, nkbs, pos_y, xt, batch_y[:, None], bx)


def _key_ranges(batch_y, batch_x, qb, kb):
    b_lo = batch_y[::qb]
    b_hi = batch_y[qb - 1::qb]
    seg_start = jnp.searchsorted(batch_x, b_lo, side="left")
    seg_end = jnp.searchsorted(batch_x, b_hi, side="right")
    kb0 = ((seg_start // 128) * 128).astype(jnp.int32)
    nkb = jnp.maximum((seg_end - kb0 + kb - 1) // kb, 1).astype(jnp.int32)
    return kb0, nkb


def _sc_maxgather(table, nbr, d_red):
    m, k = nbr.shape
    d = 128
    nw = 32
    mq = m // nw
    chunk_q = 128 // k
    chunk = chunk_q * k
    nchunks = mq // chunk_q
    cpr = min(4, nchunks)
    rounds = nchunks // cpr
    rq = cpr * chunk_q
    idx3 = nbr.reshape(nw, nchunks, chunk)
    mesh = plsc.VectorSubcoreMesh(core_axis_name="c", subcore_axis_name="s")

    @functools.partial(
        pl.kernel, mesh=mesh,
        out_type=jax.ShapeDtypeStruct((m, d), jnp.float32),
        scratch_types=[
            pltpu.VMEM((nchunks, chunk), jnp.int32),
            pltpu.VMEM((cpr * chunk, d), jnp.float32),
            pltpu.VMEM((mq, d), jnp.float32),
            pltpu.SemaphoreType.DMA,
        ],
    )
    def gather_kernel(table_hbm, idx_hbm, out_hbm, idx_v, rows_v, out_v, sem):
        wid = lax.axis_index("s") * 2 + lax.axis_index("c")
        pltpu.sync_copy(idx_hbm.at[wid], idx_v)
        for r in range(rounds):
            copies = [pltpu.async_copy(
                table_hbm.at[idx_v.at[r * cpr + g]],
                rows_v.at[pl.ds(g * chunk, chunk)],
                sem) for g in range(cpr)]
            for c in copies:
                c.wait()

            def qbody(q, carry, r=r):
                for c0 in range(d_red // 16):
                    acc = rows_v[q * k, pl.ds(c0 * 16, 16)]
                    for j in range(1, k):
                        acc = jnp.maximum(acc,
                                          rows_v[q * k + j, pl.ds(c0 * 16, 16)])
                    out_v[r * rq + q, pl.ds(c0 * 16, 16)] = acc
                return carry

            lax.fori_loop(0, rq, qbody, 0)
        pltpu.sync_copy(out_v, out_hbm.at[pl.ds(wid * mq, mq)])

    return gather_kernel(table, idx3)


def _mm3(p, w):
    return p[:, 0:1] * w[0:1, :] + p[:, 1:2] * w[1:2, :] + p[:, 2:3] * w[2:3, :]


def _prep1_kernel(pos_ref, w1_ref, p_ref):
    p_ref[...] = _mm3(pos_ref[...], w1_ref[...])


def _prep2_kernel(g1_ref, pos1_ref, pos2_ref, w1_ref, b1_ref, w2a_ref,
                  w2b_ref, b2_ref, q_ref, c2_ref):
    pos1 = pos1_ref[...]
    x1 = g1_ref[...] + (b1_ref[...] - _mm3(pos1, w1_ref[...]))
    w2b = w2b_ref[...]
    r = _mm3(pos1, w2b)
    q_ref[...] = jnp.dot(x1, w2a_ref[...],
                         preferred_element_type=jnp.float32) + r
    c2_ref[...] = b2_ref[...] - _mm3(pos2_ref[...], w2b)


def _final_kernel(g2_ref, c2_ref, pos2_ref, bm_ref, w3ax_ref, w3ap_ref,
                  b3a_ref, w3b_ref, b3b_ref, w3c_ref, b3c_ref, w4a_ref,
                  b4a_ref, w4b_ref, b4b_ref, w4c_ref, b4c_ref, out_ref):
    x2 = g2_ref[...] + c2_ref[...]
    hp = _mm3(pos2_ref[...], w3ap_ref[...])
    h = jnp.maximum(jnp.dot(x2, w3ax_ref[...],
                            preferred_element_type=jnp.float32)
                    + hp + b3a_ref[...], 0.0)
    h = jnp.maximum(jnp.dot(h, w3b_ref[...],
                            preferred_element_type=jnp.float32)
                    + b3b_ref[...], 0.0)
    h = jnp.dot(h, w3c_ref[...], preferred_element_type=jnp.float32) \
        + b3c_ref[...]
    bm = bm_ref[...]
    rows = [jnp.max(jnp.where(bm == b, h, -jnp.inf), axis=0, keepdims=True)
            for b in range(_NB)]
    g = jnp.concatenate(rows, axis=0)
    o = jnp.maximum(jnp.dot(g, w4a_ref[...],
                            preferred_element_type=jnp.float32)
                    + b4a_ref[...], 0.0)
    o = jnp.maximum(jnp.dot(o, w4b_ref[...],
                            preferred_element_type=jnp.float32)
                    + b4b_ref[...], 0.0)
    o = jnp.dot(o, w4c_ref[...], preferred_element_type=jnp.float32) \
        + b4c_ref[...]
    e = o - jnp.max(o, axis=1, keepdims=True)
    out_ref[...] = e - jnp.log(jnp.sum(jnp.exp(e), axis=1, keepdims=True))


def _call(kern, out_shapes, *args):
    return pl.pallas_call(kern, out_shape=out_shapes)(*args)


def kernel(pos, batch, W1, b1, W2, b2, W3a, b3a, W3b, b3b, W3c, b3c,
           W4a, b4a, W4b, b4b, W4c, b4c):
    batch = batch.astype(jnp.int32)
    pos1 = pos[::_DEC]
    batch1 = batch[::_DEC]
    pos2 = pos1[::_DEC]
    batch2 = batch1[::_DEC]

    kb0s1, nkbs1 = _key_ranges(batch1, batch, qb=256, kb=2304)
    nbr1 = _knn(pos1, batch1, pos, batch, kb0s1, nkbs1, qb=256, kb=2304)
    w1p = jnp.pad(W1, ((0, 0), (0, 96)))
    p_feat = _call(_prep1_kernel, jax.ShapeDtypeStruct((pos.shape[0], 128),
                                                       jnp.float32), pos, w1p)
    g1 = _sc_maxgather(p_feat, nbr1, d_red=32)[:, :32]

    q_feat, c2 = _call(
        _prep2_kernel,
        (jax.ShapeDtypeStruct((pos1.shape[0], 128), jnp.float32),
         jax.ShapeDtypeStruct((pos2.shape[0], 128), jnp.float32)),
        g1, pos1, pos2, W1, b1[None, :], W2[:32], W2[32:], b2[None, :])

    kb0s2, nkbs2 = _key_ranges(batch2, batch1, qb=256, kb=768)
    nbr2 = _knn(pos2, batch2, pos1, batch1, kb0s2, nkbs2, qb=256, kb=768)
    g2 = _sc_maxgather(q_feat, nbr2, d_red=128)

    return _call(
        _final_kernel, jax.ShapeDtypeStruct((_NB, 10), jnp.float32),
        g2, c2, pos2, batch2[:, None], W3a[:128], W3a[128:], b3a[None, :],
        W3b, b3b[None, :], W3c, b3c[None, :], W4a, b4a[None, :],
        W4b, b4b[None, :], W4c, b4c[None, :])
